# baseline (speedup 1.0000x reference)
import jax
import jax.numpy as jnp
from jax import lax
from jax.experimental import pallas as pl
from jax.experimental.pallas import tpu as pltpu
from jax.experimental.pallas import tpu_sc as plsc

NUM_FIELDS = 26
FIELD_DIM = 40000
EMBED_DIM = 16
BATCH = 16384

NC, NS = 2, 16
NW = NC * NS
TOTAL = BATCH * NUM_FIELDS
PER_W = TOTAL // NW
BPW = BATCH // NW
NCH = 8
BCH = BPW // NCH
CH = BCH * NUM_FIELDS
NBH = BATCH // 128


VOCAB = NUM_FIELDS * FIELD_DIM
NVH = VOCAB // 128
VHW = 254
NSB = 32


def _relayout_body(wq_hbm, w_hbm, tin0, tin1, to0, to1, rs0, rs1, ws0, ws1):
    wid = lax.axis_index("s") * NC + lax.axis_index("c")
    lo = wid * VHW
    hi = lax.min(NVH, lo + VHW)
    tins = (tin0, tin1)
    touts = (to0, to1)
    rsems = (rs0, rs1)
    wsems = (ws0, ws1)

    iota = lax.iota(jnp.int32, 16)
    zero = iota * 0
    cole = [zero + e for e in range(EMBED_DIM)]

    def base_of(j):
        return lax.min(lo + j * 8, hi - 8)

    def issue_reads(j, b):
        base = base_of(j)
        return (
            pltpu.async_copy(wq_hbm.at[0, pl.ds(base, 8)], tins[b].at[0], rsems[b]),
            pltpu.async_copy(wq_hbm.at[1, pl.ds(base, 8)], tins[b].at[1], rsems[b]),
        )

    def transpose(b):
        def jbody(j, _):
            vhr = lax.shift_right_logical(j, 3)
            k = lax.bitwise_and(j, 7)
            rowv = iota + (vhr * 128 + k * 16)
            for ehi in range(2):
                for elo in range(8):
                    vals = tins[b][ehi, vhr, elo, pl.ds(k * 16, 16)]
                    plsc.store_scatter(
                        touts[b], [rowv, cole[ehi * 8 + elo]], vals)
            return 0
        lax.fori_loop(0, 64, jbody, 0)

    def issue_write(j, b):
        return pltpu.async_copy(
            touts[b].at[:, pl.ds(0, EMBED_DIM)],
            w_hbm.at[pl.ds(base_of(j) * 128, 1024)], wsems[b])

    rcps = [issue_reads(0, 0), None]
    wcps = [None, None]
    for j in range(NSB):
        b = j % 2
        if j + 1 < NSB:
            rcps[(j + 1) % 2] = issue_reads(j + 1, (j + 1) % 2)
        rcps[b][0].wait()
        rcps[b][1].wait()
        if wcps[b] is not None:
            wcps[b].wait()
        transpose(b)
        wcps[b] = issue_write(j, b)
    wcps[0].wait()
    wcps[1].wait()


def _gather_body(table_hbm, xt_hbm, out5_hbm,
                 xv, idx0, idx1, rows0, rows1, ob0, ob1,
                 semx, sem0, sem1):
    wid = lax.axis_index("s") * NC + lax.axis_index("c")
    b0 = wid * BPW
    idx_b = (idx0, idx1)
    rows_b = (rows0, rows1)
    ob_b = (ob0, ob1)
    sems = (sem0, sem1)

    xcps = [
        pltpu.async_copy(xt_hbm.at[pl.ds(f * BATCH + b0, BPW)], xv.at[f], semx)
        for f in range(NUM_FIELDS)
    ]
    for c in xcps:
        c.wait()

    iota = lax.iota(jnp.int32, 16)
    iota_f = iota * NUM_FIELDS

    def build_idx(s, b):
        def bodyk(k, _):
            for f in range(NUM_FIELDS):
                vals = xv[f, pl.ds(s * BCH + k * 16, 16)]
                pos = iota_f + (k * 16 * NUM_FIELDS + f)
                plsc.store_scatter(idx_b[b], [pos], vals + f * FIELD_DIM)
            return 0
        lax.fori_loop(0, BCH // 16, bodyk, 0)

    def load_chunk(s, b):
        build_idx(s, b)
        return pltpu.async_copy(table_hbm.at[idx_b[b]], rows_b[b], sems[b])

    zero = iota * 0

    def transpose_chunk(b):
        def bodym(m, _):
            e = lax.shift_right_logical(m, 2)
            k = lax.bitwise_and(m, BCH // 16 - 1)
            ehi = lax.shift_right_logical(e, 3)
            elo = lax.bitwise_and(e, 7)
            col = zero + e
            rowk = iota_f + k * (16 * NUM_FIELDS)
            for f in range(NUM_FIELDS):
                vals = plsc.load_gather(rows_b[b], [rowk + f, col])
                ob_b[b][f, ehi, elo, pl.ds(k * 16, 16)] = vals
            return 0
        lax.fori_loop(0, EMBED_DIM * (BCH // 16), bodym, 0)

    def store_chunk(s, b):
        transpose_chunk(b)
        bhi = wid * (BPW // 128) + s // 2
        half = s % 2
        pltpu.sync_copy(
            ob_b[b],
            out5_hbm.at[:, :, bhi, :, pl.ds(half * BCH, BCH)])

    cps = [None, None]
    cps[0] = load_chunk(0, 0)
    for s in range(1, NCH):
        cps[s % 2] = load_chunk(s, s % 2)
        cps[(s - 1) % 2].wait()
        store_chunk(s - 1, (s - 1) % 2)
    cps[(NCH - 1) % 2].wait()
    store_chunk(NCH - 1, (NCH - 1) % 2)


@jax.jit
def kernel(x, weight):
    mesh = plsc.VectorSubcoreMesh(core_axis_name="c", subcore_axis_name="s")
    wq = weight.T.reshape(2, 8, NVH, 128).transpose(0, 2, 1, 3)
    w_lin = pl.kernel(
        _relayout_body,
        out_type=jax.ShapeDtypeStruct((VOCAB, EMBED_DIM), jnp.float32),
        mesh=mesh,
        scratch_types=[
            pltpu.VMEM((2, 8, 8, 128), jnp.float32),
            pltpu.VMEM((2, 8, 8, 128), jnp.float32),
            pltpu.VMEM((1024, EMBED_DIM + 1), jnp.float32),
            pltpu.VMEM((1024, EMBED_DIM + 1), jnp.float32),
            pltpu.SemaphoreType.DMA,
            pltpu.SemaphoreType.DMA,
            pltpu.SemaphoreType.DMA,
            pltpu.SemaphoreType.DMA,
        ],
        compiler_params=pltpu.CompilerParams(
            use_tc_tiling_on_sc=False, needs_layout_passes=False),
    )(wq)
    out5 = pl.kernel(
        _gather_body,
        out_type=jax.ShapeDtypeStruct(
            (NUM_FIELDS, 2, NBH, 8, 128), jnp.float32),
        mesh=mesh,
        scratch_types=[
            pltpu.VMEM((NUM_FIELDS, BPW), jnp.int32),
            pltpu.VMEM((CH,), jnp.int32),
            pltpu.VMEM((CH,), jnp.int32),
            pltpu.VMEM((CH, EMBED_DIM), jnp.float32),
            pltpu.VMEM((CH, EMBED_DIM), jnp.float32),
            pltpu.VMEM((NUM_FIELDS, 2, 8, BCH), jnp.float32),
            pltpu.VMEM((NUM_FIELDS, 2, 8, BCH), jnp.float32),
            pltpu.SemaphoreType.DMA,
            pltpu.SemaphoreType.DMA,
            pltpu.SemaphoreType.DMA,
        ],
        compiler_params=pltpu.CompilerParams(
            use_tc_tiling_on_sc=False, needs_layout_passes=False),
    )(w_lin, x.T.reshape(TOTAL))
    return out5.transpose(2, 4, 0, 1, 3).reshape(BATCH, NUM_FIELDS, EMBED_DIM)


# speedup vs baseline: 1.5008x; 1.5008x over previous
import jax
import jax.numpy as jnp
from jax import lax
from jax.experimental import pallas as pl
from jax.experimental.pallas import tpu as pltpu
from jax.experimental.pallas import tpu_sc as plsc

NUM_FIELDS = 26
FIELD_DIM = 40000
EMBED_DIM = 16
BATCH = 16384

NC, NS = 2, 16
NW = NC * NS
TOTAL = BATCH * NUM_FIELDS
PER_W = TOTAL // NW
BPW = BATCH // NW
NCH = 8
BCH = BPW // NCH
CH = BCH * NUM_FIELDS
NBH = BATCH // 128


VOCAB = NUM_FIELDS * FIELD_DIM
NVH = VOCAB // 128
VHW = 254
NSB = 32


def _relayout_body(wq_hbm, w_hbm, tin0, tin1, to0, to1, rs0, rs1, ws0, ws1):
    wid = lax.axis_index("s") * NC + lax.axis_index("c")
    lo = wid * VHW
    hi = lax.min(NVH, lo + VHW)
    tins = (tin0, tin1)
    touts = (to0, to1)
    rsems = (rs0, rs1)
    wsems = (ws0, ws1)

    iota = lax.iota(jnp.int32, 16)
    pose = [iota * EMBED_DIM + e for e in range(EMBED_DIM)]

    def base_of(j):
        return lax.min(lo + j * 8, hi - 8)

    def issue_reads(j, b):
        base = base_of(j)
        return (
            pltpu.async_copy(wq_hbm.at[0, pl.ds(base, 8)], tins[b].at[0], rsems[b]),
            pltpu.async_copy(wq_hbm.at[1, pl.ds(base, 8)], tins[b].at[1], rsems[b]),
        )

    def transpose(b):
        def jbody(j, _):
            vhr = lax.shift_right_logical(j, 3)
            k = lax.bitwise_and(j, 7)
            vbase = vhr * 2048 + k * 256
            for ehi in range(2):
                for elo in range(8):
                    vals = tins[b][ehi, vhr, elo, pl.ds(k * 16, 16)]
                    plsc.store_scatter(
                        touts[b], [pose[ehi * 8 + elo] + vbase], vals)
            return 0
        lax.fori_loop(0, 64, jbody, 0)

    def issue_write(j, b):
        return pltpu.async_copy(
            touts[b], w_hbm.at[pl.ds(base_of(j) * 2048, 16384)], wsems[b])

    rcps = [issue_reads(0, 0), None]
    wcps = [None, None]
    for j in range(NSB):
        b = j % 2
        if j + 1 < NSB:
            rcps[(j + 1) % 2] = issue_reads(j + 1, (j + 1) % 2)
        rcps[b][0].wait()
        rcps[b][1].wait()
        if wcps[b] is not None:
            wcps[b].wait()
        transpose(b)
        wcps[b] = issue_write(j, b)
    wcps[0].wait()
    wcps[1].wait()


def _gather_body(table_hbm, xt_hbm, out5_hbm,
                 xv, idx0, idx1, rows0, rows1, ob0, ob1,
                 semx, sem0, sem1):
    wid = lax.axis_index("s") * NC + lax.axis_index("c")
    b0 = wid * BPW
    idx_b = (idx0, idx1)
    rows_b = (rows0, rows1)
    ob_b = (ob0, ob1)
    sems = (sem0, sem1)

    xcps = [
        pltpu.async_copy(xt_hbm.at[pl.ds(f * BATCH + b0, BPW)], xv.at[f], semx)
        for f in range(NUM_FIELDS)
    ]
    for c in xcps:
        c.wait()

    iota = lax.iota(jnp.int32, 16)
    iota_f = iota * NUM_FIELDS

    def build_idx(s, b):
        def bodyk(k, _):
            for f in range(NUM_FIELDS):
                vals = xv[f, pl.ds(s * BCH + k * 16, 16)]
                pos = iota_f + (k * 16 * NUM_FIELDS + f)
                plsc.store_scatter(idx_b[b], [pos], vals + f * FIELD_DIM)
            return 0
        lax.fori_loop(0, BCH // 16, bodyk, 0)

    def load_chunk(s, b):
        build_idx(s, b)
        return pltpu.async_copy(table_hbm.at[idx_b[b]], rows_b[b], sems[b])

    zero = iota * 0

    def transpose_chunk(b):
        def bodym(m, _):
            e = lax.shift_right_logical(m, 2)
            k = lax.bitwise_and(m, BCH // 16 - 1)
            ehi = lax.shift_right_logical(e, 3)
            elo = lax.bitwise_and(e, 7)
            col = zero + e
            rowk = iota_f + k * (16 * NUM_FIELDS)
            for f in range(NUM_FIELDS):
                vals = plsc.load_gather(rows_b[b], [rowk + f, col])
                ob_b[b][f, ehi, elo, pl.ds(k * 16, 16)] = vals
            return 0
        lax.fori_loop(0, EMBED_DIM * (BCH // 16), bodym, 0)

    def store_chunk(s, b):
        transpose_chunk(b)
        bhi = wid * (BPW // 128) + s // 2
        half = s % 2
        pltpu.sync_copy(
            ob_b[b],
            out5_hbm.at[:, :, bhi, :, pl.ds(half * BCH, BCH)])

    cps = [None, None]
    cps[0] = load_chunk(0, 0)
    for s in range(1, NCH):
        cps[s % 2] = load_chunk(s, s % 2)
        cps[(s - 1) % 2].wait()
        store_chunk(s - 1, (s - 1) % 2)
    cps[(NCH - 1) % 2].wait()
    store_chunk(NCH - 1, (NCH - 1) % 2)


@jax.jit
def kernel(x, weight):
    mesh = plsc.VectorSubcoreMesh(core_axis_name="c", subcore_axis_name="s")
    wq = weight.T.reshape(2, 8, NVH, 128).transpose(0, 2, 1, 3)
    w_lin = pl.kernel(
        _relayout_body,
        out_type=jax.ShapeDtypeStruct((VOCAB * EMBED_DIM,), jnp.float32),
        mesh=mesh,
        scratch_types=[
            pltpu.VMEM((2, 8, 8, 128), jnp.float32),
            pltpu.VMEM((2, 8, 8, 128), jnp.float32),
            pltpu.VMEM((1024 * EMBED_DIM,), jnp.float32),
            pltpu.VMEM((1024 * EMBED_DIM,), jnp.float32),
            pltpu.SemaphoreType.DMA,
            pltpu.SemaphoreType.DMA,
            pltpu.SemaphoreType.DMA,
            pltpu.SemaphoreType.DMA,
        ],
        compiler_params=pltpu.CompilerParams(
            use_tc_tiling_on_sc=False, needs_layout_passes=False),
    )(wq)
    w_lin = w_lin.reshape(VOCAB, EMBED_DIM)
    out5 = pl.kernel(
        _gather_body,
        out_type=jax.ShapeDtypeStruct(
            (NUM_FIELDS, 2, NBH, 8, 128), jnp.float32),
        mesh=mesh,
        scratch_types=[
            pltpu.VMEM((NUM_FIELDS, BPW), jnp.int32),
            pltpu.VMEM((CH,), jnp.int32),
            pltpu.VMEM((CH,), jnp.int32),
            pltpu.VMEM((CH, EMBED_DIM), jnp.float32),
            pltpu.VMEM((CH, EMBED_DIM), jnp.float32),
            pltpu.VMEM((NUM_FIELDS, 2, 8, BCH), jnp.float32),
            pltpu.VMEM((NUM_FIELDS, 2, 8, BCH), jnp.float32),
            pltpu.SemaphoreType.DMA,
            pltpu.SemaphoreType.DMA,
            pltpu.SemaphoreType.DMA,
        ],
        compiler_params=pltpu.CompilerParams(
            use_tc_tiling_on_sc=False, needs_layout_passes=False),
    )(w_lin, x.T.reshape(TOTAL))
    return out5.transpose(2, 4, 0, 1, 3).reshape(BATCH, NUM_FIELDS, EMBED_DIM)
